# baseline (device time: 10801 ns/iter reference)
import jax
import jax.numpy as jnp
from jax import lax
from jax.experimental import pallas as pl
from jax.experimental.pallas import tpu as pltpu

N_DEV = 4
N_HALVES = 2
ROWS_PER_STEP = 256


def kernel(x):
    m_per, n = x.shape
    n_steps = m_per // ROWS_PER_STEP
    assert m_per % ROWS_PER_STEP == 0
    n2 = n // N_HALVES
    assert n % (N_HALVES * 128) == 0

    def body(x_ref, out_ref, acc_ref, comm_ref, send_sems, recv_sems):
        half = pl.program_id(0)
        step = pl.program_id(1)
        my = lax.axis_index("i")
        barrier_sem = pltpu.get_barrier_semaphore()

        @pl.when(jnp.logical_and(half == 0, step == 0))
        def _():
            for d in range(1, N_DEV):
                peer = lax.rem(my + d, N_DEV)
                pl.semaphore_signal(
                    barrier_sem, inc=1,
                    device_id=(peer,), device_id_type=pl.DeviceIdType.MESH,
                )

        chunk_max = jnp.max(x_ref[:, :], axis=0)

        for j in range(N_HALVES):
            col = pl.ds(j * n2, n2)

            @pl.when(jnp.logical_and(half == j, step == 0))
            def _(j=j, col=col):
                acc_ref[0, col] = chunk_max

            @pl.when(jnp.logical_and(half == j, step > 0))
            def _(j=j, col=col):
                acc_ref[0, col] = jnp.maximum(acc_ref[0, col], chunk_max)

            @pl.when(jnp.logical_and(half == j, step == n_steps - 1))
            def _(j=j, col=col):
                if j == 0:
                    pl.semaphore_wait(barrier_sem, N_DEV - 1)
                for d in range(1, N_DEV):
                    peer = lax.rem(my + d, N_DEV)
                    slot = d - 1
                    rdma = pltpu.make_async_remote_copy(
                        src_ref=acc_ref.at[:, col],
                        dst_ref=comm_ref.at[slot, :, col],
                        send_sem=send_sems.at[j, slot],
                        recv_sem=recv_sems.at[j, slot],
                        device_id=(peer,),
                        device_id_type=pl.DeviceIdType.MESH,
                    )
                    rdma.start()

        @pl.when(jnp.logical_and(half == N_HALVES - 1, step == n_steps - 1))
        def _():
            for j in range(N_HALVES):
                col = pl.ds(j * n2, n2)
                for slot in range(N_DEV - 1):
                    rdma = pltpu.make_async_remote_copy(
                        src_ref=acc_ref.at[:, col],
                        dst_ref=comm_ref.at[slot, :, col],
                        send_sem=send_sems.at[j, slot],
                        recv_sem=recv_sems.at[j, slot],
                        device_id=(lax.rem(my + slot + 1, N_DEV),),
                        device_id_type=pl.DeviceIdType.MESH,
                    )
                    rdma.wait_send()
                    rdma.wait_recv()

            acc = acc_ref[0, :]
            for slot in range(N_DEV - 1):
                acc = jnp.maximum(acc, comm_ref[slot, 0, :])
            out_ref[0, :] = acc

    return pl.pallas_call(
        body,
        grid=(N_HALVES, n_steps),
        out_shape=jax.ShapeDtypeStruct((1, n), jnp.float32),
        in_specs=[
            pl.BlockSpec(
                (ROWS_PER_STEP, n2), lambda j, i: (i, j),
                memory_space=pltpu.VMEM,
            )
        ],
        out_specs=pl.BlockSpec(
            (1, n), lambda j, i: (0, 0), memory_space=pltpu.VMEM
        ),
        scratch_shapes=[
            pltpu.VMEM((1, n), jnp.float32),
            pltpu.VMEM((N_DEV - 1, 1, n), jnp.float32),
            pltpu.SemaphoreType.DMA((N_HALVES, N_DEV - 1)),
            pltpu.SemaphoreType.DMA((N_HALVES, N_DEV - 1)),
        ],
        compiler_params=pltpu.CompilerParams(collective_id=0),
    )(x)


# device time: 9065 ns/iter; 1.1915x vs baseline; 1.1915x over previous
import jax
import jax.numpy as jnp
from jax import lax
from jax.experimental import pallas as pl
from jax.experimental.pallas import tpu as pltpu

N_DEV = 4
ROWS_PER_STEP = 1024


def kernel(x):
    m_per, n = x.shape
    n_steps = m_per // ROWS_PER_STEP
    assert m_per % ROWS_PER_STEP == 0

    def body(x_ref, out_ref, acc_ref, comm_ref, send_sems, recv_sems):
        step = pl.program_id(0)
        my = lax.axis_index("i")
        barrier_sem = pltpu.get_barrier_semaphore()

        @pl.when(step == 0)
        def _():
            for d in range(1, N_DEV):
                peer = lax.rem(my + d, N_DEV)
                pl.semaphore_signal(
                    barrier_sem, inc=1,
                    device_id=(peer,), device_id_type=pl.DeviceIdType.MESH,
                )

        chunk_max = jnp.max(x_ref[:, :], axis=0)

        @pl.when(step == 0)
        def _():
            acc_ref[0, :] = chunk_max

        @pl.when(step > 0)
        def _():
            acc_ref[0, :] = jnp.maximum(acc_ref[0, :], chunk_max)

        @pl.when(step == n_steps - 1)
        def _():
            pl.semaphore_wait(barrier_sem, N_DEV - 1)

            rdmas = []
            for d in range(1, N_DEV):
                peer = lax.rem(my + d, N_DEV)
                slot = d - 1
                rdma = pltpu.make_async_remote_copy(
                    src_ref=acc_ref,
                    dst_ref=comm_ref.at[slot],
                    send_sem=send_sems.at[slot],
                    recv_sem=recv_sems.at[slot],
                    device_id=(peer,),
                    device_id_type=pl.DeviceIdType.MESH,
                )
                rdma.start()
                rdmas.append(rdma)

            for rdma in rdmas:
                rdma.wait_send()
            for rdma in rdmas:
                rdma.wait_recv()

            acc = acc_ref[0, :]
            for slot in range(N_DEV - 1):
                acc = jnp.maximum(acc, comm_ref[slot, 0, :])
            out_ref[0, :] = acc

    return pl.pallas_call(
        body,
        grid=(n_steps,),
        out_shape=jax.ShapeDtypeStruct((1, n), jnp.float32),
        in_specs=[
            pl.BlockSpec(
                (ROWS_PER_STEP, n), lambda i: (i, 0),
                memory_space=pltpu.VMEM,
            )
        ],
        out_specs=pl.BlockSpec((1, n), lambda i: (0, 0), memory_space=pltpu.VMEM),
        scratch_shapes=[
            pltpu.VMEM((1, n), jnp.float32),
            pltpu.VMEM((N_DEV - 1, 1, n), jnp.float32),
            pltpu.SemaphoreType.DMA((N_DEV - 1,)),
            pltpu.SemaphoreType.DMA((N_DEV - 1,)),
        ],
        compiler_params=pltpu.CompilerParams(collective_id=0),
    )(x)
